# baseline (speedup 1.0000x reference)
import functools
import jax
import jax.numpy as jnp
from jax import lax
from jax.experimental import pallas as pl
from jax.experimental.pallas import tpu as pltpu
from jax.experimental.pallas import tpu_sc as plsc

N = 10000
E = 320000
D = 128
DE = 16
H = 128
DG = 128
G = 16

NC = 2
NS = 16
NW = NC * NS
B = 64
CH0 = 240
CH1 = 76
E_PAD = NS * (CH0 + CH1) * B
SROWS = 10240
RPW = SROWS // NS
EB = 6400
NBUF = 4
NB = 1024


def _xa_body(x_ref, w_ref, o_ref):
    o_ref[:] = jnp.dot(x_ref[:], w_ref[:], preferred_element_type=jnp.float32)


def _tc_xa(x, w1a):
    return pl.pallas_call(
        _xa_body,
        out_shape=jax.ShapeDtypeStruct((N, H), jnp.float32),
    )(x, w1a)


def _ea_body(e_ref, w_ref, b_ref, o_ref):
    o_ref[:] = (lax.dot_general(e_ref[:], w_ref[:],
                                (((0,), (0,)), ((), ())),
                                preferred_element_type=jnp.float32)
                + b_ref[:])


def _tc_ea(edge_attr, w1b, b1):
    nreal = E // EB
    nblk = -(-E_PAD // EB)
    return pl.pallas_call(
        _ea_body,
        grid=(nblk,),
        in_specs=[
            pl.BlockSpec((DE, EB), lambda i: (0, jnp.minimum(i, nreal - 1))),
            pl.BlockSpec((DE, H), lambda i: (0, 0)),
            pl.BlockSpec((1, H), lambda i: (0, 0)),
        ],
        out_specs=pl.BlockSpec((EB, H), lambda i: (i, 0)),
        out_shape=jax.ShapeDtypeStruct((nblk * EB, H), jnp.float32),
    )(edge_attr.T, w1b, b1.reshape(1, H))


def _sc_body(xa_hbm, ea_hbm, row_hbm, col_hbm, out_hbm, cnt_hbm,
             row_v, col_v, work_v, hist_v, acc_sh, *sems):
    cid = lax.axis_index("c")
    sid = lax.axis_index("s")
    wid = sid * NC + cid
    semA = sems[0:NBUF]
    semG = sems[NBUF:2 * NBUF]
    semS = sems[2 * NBUF:3 * NBUF]

    zero16 = jnp.zeros((16,), jnp.float32)

    def zrow(b, c):
        for j in range(H // 16):
            work_v[0, b, pl.ds(j * 16, 16)] = zero16
        return c
    lax.fori_loop(0, B, zrow, 0)
    for t in range(RPW // B):
        pltpu.sync_copy(work_v.at[0], acc_sh.at[pl.ds(sid * RPW + t * B, B)])

    def zhist(i, c):
        hist_v[pl.ds(i * 16, 16)] = zero16
        return c
    lax.fori_loop(0, SROWS // 16, zhist, 0)
    plsc.subcore_barrier()

    chc = jnp.where(cid == 0, CH0, CH1)
    ebase = jnp.where(cid == 0, sid * (CH0 * B),
                      NS * (CH0 * B) + sid * (CH1 * B))

    def a_descs(c, p):
        base = ebase + c * B
        return (
            pltpu.make_async_copy(row_hbm.at[pl.ds(base, B)], row_v.at[p],
                                  semA[p]),
            pltpu.make_async_copy(col_hbm.at[pl.ds(base, B)], col_v.at[p],
                                  semA[p]),
            pltpu.make_async_copy(ea_hbm.at[pl.ds(base, B)], work_v.at[p],
                                  semA[p]),
        )

    def issue_a(c, p):
        base = ebase + c * B
        pltpu.async_copy(row_hbm.at[pl.ds(base, B)], row_v.at[p], semA[p])
        pltpu.async_copy(col_hbm.at[pl.ds(base, B)], col_v.at[p], semA[p])
        pltpu.async_copy(ea_hbm.at[pl.ds(base, B)], work_v.at[p], semA[p])

    def wait_a(c, p):
        for d in a_descs(c, p):
            d.wait()

    def issue_g(p):
        pltpu.async_copy(xa_hbm.at[row_v.at[p]], work_v.at[p],
                         semG[p], add=True)

    def wait_g(p):
        pltpu.make_async_copy(xa_hbm.at[row_v.at[p]], work_v.at[p],
                              semG[p]).wait()

    def wait_s(p):
        pltpu.make_async_copy(work_v.at[p], acc_sh.at[col_v.at[p]],
                              semS[p]).wait()

    def relu_scatter(p):
        def relu_row(b, cc):
            for j in range(H // 16):
                v = work_v[p, b, pl.ds(j * 16, 16)]
                work_v[p, b, pl.ds(j * 16, 16)] = jnp.maximum(v, 0.0)
            return cc
        lax.fori_loop(0, B, relu_row, 0)
        pltpu.async_copy(work_v.at[p], acc_sh.at[col_v.at[p]],
                         semS[p], add=True)
        ones16 = jnp.full((16,), 1.0, jnp.float32)
        for k in range(B // 16):
            col16 = col_v[p, pl.ds(k * 16, 16)]
            plsc.addupdate_scatter(hist_v, [col16], ones16)

    issue_a(0, 0)
    issue_a(1, 1)
    issue_a(2, 2)
    wait_a(0, 0)
    issue_g(0)
    wait_a(1, 1)
    issue_g(1)

    def step(c, p):
        @pl.when(c + 2 <= chc - 1)
        def _():
            wait_a(c + 2, (p + 2) % NBUF)
            issue_g((p + 2) % NBUF)

        wait_g(p)
        relu_scatter(p)

        @pl.when(c >= 1)
        def _():
            wait_s((p - 1) % NBUF)

        @pl.when(c + 3 <= chc - 1)
        def _():
            issue_a(c + 3, (p + 3) % NBUF)

    def quad(g, carry):
        for k in range(NBUF):
            step(NBUF * g + k, k)
        return carry
    lax.fori_loop(0, chc // NBUF, quad, 0)
    wait_s((CH0 - 1) % NBUF)

    plsc.subcore_barrier()
    for t in range(RPW // B):
        r0 = sid * RPW + t * B
        pltpu.sync_copy(acc_sh.at[pl.ds(r0, B)], out_hbm.at[cid, pl.ds(r0, B)])
    pltpu.sync_copy(hist_v, cnt_hbm.at[wid])


def _sc_scatter(xa, ea_pad, row_pad, col_pad):
    mesh = plsc.VectorSubcoreMesh(core_axis_name="c", subcore_axis_name="s")
    kern = pl.kernel(
        _sc_body,
        out_type=(
            jax.ShapeDtypeStruct((NC, SROWS, H), jnp.float32),
            jax.ShapeDtypeStruct((NW, SROWS), jnp.float32),
        ),
        mesh=mesh,
        scratch_types=[
            pltpu.VMEM((NBUF, B), jnp.int32),
            pltpu.VMEM((NBUF, B), jnp.int32),
            pltpu.VMEM((NBUF, B, H), jnp.float32),
            pltpu.VMEM((SROWS,), jnp.float32),
            pltpu.VMEM_SHARED((SROWS, H), jnp.float32),
        ] + [pltpu.SemaphoreType.DMA] * (3 * NBUF),
        compiler_params=pltpu.CompilerParams(needs_layout_passes=False),
    )
    return kern(xa, ea_pad, row_pad, col_pad)


def _post_body(s_ref, c_ref, x_ref, b_ref, u_ref, w2_ref, b2_ref,
               w3a_ref, w3b_ref, w3c_ref, b3_ref, w4_ref, b4_ref, o_ref):
    ssum = s_ref[0] + s_ref[1]
    cnt = jnp.sum(c_ref[:], axis=0)[:, None]
    mean_in = ssum / jnp.maximum(cnt, 1.0)
    mask = (cnt > 0.0).astype(jnp.float32)
    mh = (jnp.dot(mean_in, w2_ref[:], preferred_element_type=jnp.float32)
          + b2_ref[:] * mask)
    uw = jnp.dot(u_ref[:], w3c_ref[:], preferred_element_type=jnp.float32)
    bidx = b_ref[0, 0, :]
    onehot = (bidx[:, None] == lax.iota(jnp.int32, G)[None, :])
    onehot = onehot.astype(jnp.float32)
    pre = (jnp.dot(x_ref[:], w3a_ref[:], preferred_element_type=jnp.float32)
           + jnp.dot(mh, w3b_ref[:], preferred_element_type=jnp.float32)
           + jnp.dot(onehot, uw, preferred_element_type=jnp.float32)
           + b3_ref[:])
    o_ref[:] = (jnp.dot(jnp.maximum(pre, 0.0), w4_ref[:],
                        preferred_element_type=jnp.float32) + b4_ref[:])


def _tc_post(s_acc, cnt_acc, x_pad, batch_pad, u,
             w2, b2, w3a, w3b, w3c, b3, w4, b4):
    grid = SROWS // NB
    return pl.pallas_call(
        _post_body,
        grid=(grid,),
        in_specs=[
            pl.BlockSpec((NC, NB, H), lambda i: (0, i, 0)),
            pl.BlockSpec((NW, NB), lambda i: (0, i)),
            pl.BlockSpec((NB, D), lambda i: (i, 0)),
            pl.BlockSpec((1, 1, NB), lambda i: (i, 0, 0)),
            pl.BlockSpec((G, DG), lambda i: (0, 0)),
            pl.BlockSpec((H, H), lambda i: (0, 0)),
            pl.BlockSpec((1, H), lambda i: (0, 0)),
            pl.BlockSpec((D, H), lambda i: (0, 0)),
            pl.BlockSpec((H, H), lambda i: (0, 0)),
            pl.BlockSpec((DG, H), lambda i: (0, 0)),
            pl.BlockSpec((1, H), lambda i: (0, 0)),
            pl.BlockSpec((H, D), lambda i: (0, 0)),
            pl.BlockSpec((1, D), lambda i: (0, 0)),
        ],
        out_specs=pl.BlockSpec((NB, D), lambda i: (i, 0)),
        out_shape=jax.ShapeDtypeStruct((SROWS, D), jnp.float32),
    )(s_acc, cnt_acc, x_pad, batch_pad, u, w2, b2.reshape(1, H),
      w3a, w3b, w3c, b3.reshape(1, H), w4, b4.reshape(1, D))


@jax.jit
def kernel(x, edge_index, edge_attr, u, batch, W1, b1, W2, b2, W3, b3, W4, b4):
    row = edge_index[0].astype(jnp.int32)
    col = edge_index[1].astype(jnp.int32)
    w1a, w1b = W1[:D], W1[D:]
    w3a, w3b, w3c = W3[:D], W3[D:D + H], W3[D + H:]

    xa = _tc_xa(x, w1a)

    pad = E_PAD - E
    ea = _tc_ea(edge_attr, w1b, b1)
    row_pad = jnp.pad(row, (0, pad))
    col_pad = jnp.pad(col, (0, pad), constant_values=N)

    s_acc, cnt_acc = _sc_scatter(xa, ea, row_pad, col_pad)

    x_pad = jnp.pad(x, ((0, SROWS - N), (0, 0)))
    batch_pad = jnp.pad(batch.astype(jnp.int32), (0, SROWS - N))
    batch_pad = batch_pad.reshape(SROWS // NB, 1, NB)

    out = _tc_post(s_acc, cnt_acc, x_pad, batch_pad, u,
                   W2, b2, w3a, w3b, w3c, b3, W4, b4)
    return out[:N]


# speedup vs baseline: 1.0331x; 1.0331x over previous
import functools
import jax
import jax.numpy as jnp
from jax import lax
from jax.experimental import pallas as pl
from jax.experimental.pallas import tpu as pltpu
from jax.experimental.pallas import tpu_sc as plsc

N = 10000
E = 320000
D = 128
DE = 16
H = 128
DG = 128
G = 16

NC = 2
NS = 16
NW = NC * NS
B = 64
CH0 = 228
CH1 = 88
E_PAD = NS * (CH0 + CH1) * B
SROWS = 10240
RPW = SROWS // NS
EB = 6400
NBUF = 4
NB = 1024


def _xa_body(x_ref, w_ref, o_ref):
    o_ref[:] = jnp.dot(x_ref[:], w_ref[:], preferred_element_type=jnp.float32)


def _tc_xa(x, w1a):
    return pl.pallas_call(
        _xa_body,
        out_shape=jax.ShapeDtypeStruct((N, H), jnp.float32),
    )(x, w1a)


def _ea_body(e_ref, w_ref, b_ref, o_ref):
    o_ref[:] = (lax.dot_general(e_ref[:], w_ref[:],
                                (((0,), (0,)), ((), ())),
                                preferred_element_type=jnp.float32)
                + b_ref[:])


def _tc_ea(edge_attr, w1b, b1):
    nreal = E // EB
    nblk = -(-E_PAD // EB)
    return pl.pallas_call(
        _ea_body,
        grid=(nblk,),
        in_specs=[
            pl.BlockSpec((DE, EB), lambda i: (0, jnp.minimum(i, nreal - 1))),
            pl.BlockSpec((DE, H), lambda i: (0, 0)),
            pl.BlockSpec((1, H), lambda i: (0, 0)),
        ],
        out_specs=pl.BlockSpec((EB, H), lambda i: (i, 0)),
        out_shape=jax.ShapeDtypeStruct((nblk * EB, H), jnp.float32),
    )(edge_attr.T, w1b, b1.reshape(1, H))


def _sc_body(xa_hbm, ea_hbm, row_hbm, col_hbm, out_hbm, cnt_hbm,
             row_v, col_v, work_v, hist_v, acc_sh, *sems):
    cid = lax.axis_index("c")
    sid = lax.axis_index("s")
    wid = sid * NC + cid
    semA = sems[0:NBUF]
    semG = sems[NBUF:2 * NBUF]
    semS = sems[2 * NBUF:3 * NBUF]

    zero16 = jnp.zeros((16,), jnp.float32)

    def zrow(b, c):
        for j in range(H // 16):
            work_v[0, b, pl.ds(j * 16, 16)] = zero16
        return c
    lax.fori_loop(0, B, zrow, 0)
    for t in range(RPW // B):
        pltpu.sync_copy(work_v.at[0], acc_sh.at[pl.ds(sid * RPW + t * B, B)])

    def zhist(i, c):
        hist_v[pl.ds(i * 16, 16)] = zero16
        return c
    lax.fori_loop(0, SROWS // 16, zhist, 0)
    plsc.subcore_barrier()

    chc = jnp.where(cid == 0, CH0, CH1)
    ebase = jnp.where(cid == 0, sid * (CH0 * B),
                      NS * (CH0 * B) + sid * (CH1 * B))

    def a_descs(c, p):
        base = ebase + c * B
        return (
            pltpu.make_async_copy(row_hbm.at[pl.ds(base, B)], row_v.at[p],
                                  semA[p]),
            pltpu.make_async_copy(col_hbm.at[pl.ds(base, B)], col_v.at[p],
                                  semA[p]),
            pltpu.make_async_copy(ea_hbm.at[pl.ds(base, B)], work_v.at[p],
                                  semA[p]),
        )

    def issue_a(c, p):
        base = ebase + c * B
        pltpu.async_copy(row_hbm.at[pl.ds(base, B)], row_v.at[p], semA[p])
        pltpu.async_copy(col_hbm.at[pl.ds(base, B)], col_v.at[p], semA[p])
        pltpu.async_copy(ea_hbm.at[pl.ds(base, B)], work_v.at[p], semA[p])

    def wait_a(c, p):
        for d in a_descs(c, p):
            d.wait()

    def issue_g(p):
        pltpu.async_copy(xa_hbm.at[row_v.at[p]], work_v.at[p],
                         semG[p], add=True)

    def wait_g(p):
        pltpu.make_async_copy(xa_hbm.at[row_v.at[p]], work_v.at[p],
                              semG[p]).wait()

    def wait_s(p):
        pltpu.make_async_copy(work_v.at[p], acc_sh.at[col_v.at[p]],
                              semS[p]).wait()

    def relu_scatter(p):
        def relu_row(b, cc):
            for j in range(H // 16):
                v = work_v[p, b, pl.ds(j * 16, 16)]
                work_v[p, b, pl.ds(j * 16, 16)] = jnp.maximum(v, 0.0)
            return cc
        lax.fori_loop(0, B, relu_row, 0)
        pltpu.async_copy(work_v.at[p], acc_sh.at[col_v.at[p]],
                         semS[p], add=True)
        ones16 = jnp.full((16,), 1.0, jnp.float32)
        for k in range(B // 16):
            col16 = col_v[p, pl.ds(k * 16, 16)]
            plsc.addupdate_scatter(hist_v, [col16], ones16)

    issue_a(0, 0)
    issue_a(1, 1)
    issue_a(2, 2)
    wait_a(0, 0)
    issue_g(0)
    wait_a(1, 1)
    issue_g(1)

    def step(c, p):
        @pl.when(c + 2 <= chc - 1)
        def _():
            wait_a(c + 2, (p + 2) % NBUF)
            issue_g((p + 2) % NBUF)

        wait_g(p)
        relu_scatter(p)

        @pl.when(c >= 1)
        def _():
            wait_s((p - 1) % NBUF)

        @pl.when(c + 3 <= chc - 1)
        def _():
            issue_a(c + 3, (p + 3) % NBUF)

    def quad(g, carry):
        for k in range(NBUF):
            step(NBUF * g + k, k)
        return carry
    lax.fori_loop(0, chc // NBUF, quad, 0)
    wait_s((CH0 - 1) % NBUF)

    plsc.subcore_barrier()
    for t in range(RPW // B):
        r0 = sid * RPW + t * B
        pltpu.sync_copy(acc_sh.at[pl.ds(r0, B)], out_hbm.at[cid, pl.ds(r0, B)])
    pltpu.sync_copy(hist_v, cnt_hbm.at[wid])


def _sc_scatter(xa, ea_pad, row_pad, col_pad):
    mesh = plsc.VectorSubcoreMesh(core_axis_name="c", subcore_axis_name="s")
    kern = pl.kernel(
        _sc_body,
        out_type=(
            jax.ShapeDtypeStruct((NC, SROWS, H), jnp.float32),
            jax.ShapeDtypeStruct((NW, SROWS), jnp.float32),
        ),
        mesh=mesh,
        scratch_types=[
            pltpu.VMEM((NBUF, B), jnp.int32),
            pltpu.VMEM((NBUF, B), jnp.int32),
            pltpu.VMEM((NBUF, B, H), jnp.float32),
            pltpu.VMEM((SROWS,), jnp.float32),
            pltpu.VMEM_SHARED((SROWS, H), jnp.float32),
        ] + [pltpu.SemaphoreType.DMA] * (3 * NBUF),
        compiler_params=pltpu.CompilerParams(needs_layout_passes=False),
    )
    return kern(xa, ea_pad, row_pad, col_pad)


def _post_body(s_ref, c_ref, x_ref, b_ref, u_ref, w2_ref, b2_ref,
               w3a_ref, w3b_ref, w3c_ref, b3_ref, w4_ref, b4_ref, o_ref):
    ssum = s_ref[0] + s_ref[1]
    cnt = jnp.sum(c_ref[:], axis=0)[:, None]
    mean_in = ssum / jnp.maximum(cnt, 1.0)
    mask = (cnt > 0.0).astype(jnp.float32)
    mh = (jnp.dot(mean_in, w2_ref[:], preferred_element_type=jnp.float32)
          + b2_ref[:] * mask)
    uw = jnp.dot(u_ref[:], w3c_ref[:], preferred_element_type=jnp.float32)
    bidx = b_ref[0, 0, :]
    onehot = (bidx[:, None] == lax.iota(jnp.int32, G)[None, :])
    onehot = onehot.astype(jnp.float32)
    pre = (jnp.dot(x_ref[:], w3a_ref[:], preferred_element_type=jnp.float32)
           + jnp.dot(mh, w3b_ref[:], preferred_element_type=jnp.float32)
           + jnp.dot(onehot, uw, preferred_element_type=jnp.float32)
           + b3_ref[:])
    o_ref[:] = (jnp.dot(jnp.maximum(pre, 0.0), w4_ref[:],
                        preferred_element_type=jnp.float32) + b4_ref[:])


def _tc_post(s_acc, cnt_acc, x_pad, batch_pad, u,
             w2, b2, w3a, w3b, w3c, b3, w4, b4):
    grid = SROWS // NB
    return pl.pallas_call(
        _post_body,
        grid=(grid,),
        in_specs=[
            pl.BlockSpec((NC, NB, H), lambda i: (0, i, 0)),
            pl.BlockSpec((NW, NB), lambda i: (0, i)),
            pl.BlockSpec((NB, D), lambda i: (i, 0)),
            pl.BlockSpec((1, 1, NB), lambda i: (i, 0, 0)),
            pl.BlockSpec((G, DG), lambda i: (0, 0)),
            pl.BlockSpec((H, H), lambda i: (0, 0)),
            pl.BlockSpec((1, H), lambda i: (0, 0)),
            pl.BlockSpec((D, H), lambda i: (0, 0)),
            pl.BlockSpec((H, H), lambda i: (0, 0)),
            pl.BlockSpec((DG, H), lambda i: (0, 0)),
            pl.BlockSpec((1, H), lambda i: (0, 0)),
            pl.BlockSpec((H, D), lambda i: (0, 0)),
            pl.BlockSpec((1, D), lambda i: (0, 0)),
        ],
        out_specs=pl.BlockSpec((NB, D), lambda i: (i, 0)),
        out_shape=jax.ShapeDtypeStruct((SROWS, D), jnp.float32),
    )(s_acc, cnt_acc, x_pad, batch_pad, u, w2, b2.reshape(1, H),
      w3a, w3b, w3c, b3.reshape(1, H), w4, b4.reshape(1, D))


@jax.jit
def kernel(x, edge_index, edge_attr, u, batch, W1, b1, W2, b2, W3, b3, W4, b4):
    row = edge_index[0].astype(jnp.int32)
    col = edge_index[1].astype(jnp.int32)
    w1a, w1b = W1[:D], W1[D:]
    w3a, w3b, w3c = W3[:D], W3[D:D + H], W3[D + H:]

    xa = _tc_xa(x, w1a)

    pad = E_PAD - E
    ea = _tc_ea(edge_attr, w1b, b1)
    row_pad = jnp.pad(row, (0, pad))
    col_pad = jnp.pad(col, (0, pad), constant_values=N)

    s_acc, cnt_acc = _sc_scatter(xa, ea, row_pad, col_pad)

    x_pad = jnp.pad(x, ((0, SROWS - N), (0, 0)))
    batch_pad = jnp.pad(batch.astype(jnp.int32), (0, SROWS - N))
    batch_pad = batch_pad.reshape(SROWS // NB, 1, NB)

    out = _tc_post(s_acc, cnt_acc, x_pad, batch_pad, u,
                   W2, b2, w3a, w3b, w3c, b3, W4, b4)
    return out[:N]


# speedup vs baseline: 1.0503x; 1.0167x over previous
import functools
import jax
import jax.numpy as jnp
from jax import lax
from jax.experimental import pallas as pl
from jax.experimental.pallas import tpu as pltpu
from jax.experimental.pallas import tpu_sc as plsc

N = 10000
E = 320000
D = 128
DE = 16
H = 128
DG = 128
G = 16

NC = 2
NS = 16
NW = NC * NS
B = 64
CH0 = 224
CH1 = 92
E_PAD = NS * (CH0 + CH1) * B
SROWS = 10240
RPW = SROWS // NS
EB = 6400
NBUF = 4
NB = 1024


def _xa_body(x_ref, w_ref, o_ref):
    o_ref[:] = jnp.dot(x_ref[:], w_ref[:], preferred_element_type=jnp.float32)


def _tc_xa(x, w1a):
    return pl.pallas_call(
        _xa_body,
        out_shape=jax.ShapeDtypeStruct((N, H), jnp.float32),
    )(x, w1a)


def _ea_body(e_ref, w_ref, b_ref, o_ref):
    o_ref[:] = (lax.dot_general(e_ref[:], w_ref[:], (((0,), (0,)), ((), ())),
                                preferred_element_type=jnp.float32)
                + b_ref[:])


def _tc_ea(edge_attr, w1b, b1):
    nreal = E // EB
    nblk = -(-E_PAD // EB)
    return pl.pallas_call(
        _ea_body,
        grid=(nblk,),
        in_specs=[
            pl.BlockSpec((DE, EB), lambda i: (0, jnp.minimum(i, nreal - 1))),
            pl.BlockSpec((DE, H), lambda i: (0, 0)),
            pl.BlockSpec((1, H), lambda i: (0, 0)),
        ],
        out_specs=pl.BlockSpec((EB, H), lambda i: (i, 0)),
        out_shape=jax.ShapeDtypeStruct((nblk * EB, H), jnp.float32),
    )(edge_attr.T, w1b, b1.reshape(1, H))


def _sc_body(xa_hbm, ea_hbm, row_hbm, col_hbm, out_hbm, cnt_hbm,
             row_v, col_v, work_v, hist_v, acc_sh, *sems):
    cid = lax.axis_index("c")
    sid = lax.axis_index("s")
    wid = sid * NC + cid
    semA = sems[0:NBUF]
    semG = sems[NBUF:2 * NBUF]
    semS = sems[2 * NBUF:3 * NBUF]

    zero16 = jnp.zeros((16,), jnp.float32)

    def zrow(b, c):
        for j in range(H // 16):
            work_v[0, b, pl.ds(j * 16, 16)] = zero16
        return c
    lax.fori_loop(0, B, zrow, 0)
    for t in range(RPW // B):
        pltpu.sync_copy(work_v.at[0], acc_sh.at[pl.ds(sid * RPW + t * B, B)])
    rem = RPW % B
    if rem:
        pltpu.sync_copy(
            work_v.at[0, pl.ds(0, rem)],
            acc_sh.at[pl.ds(sid * RPW + (RPW // B) * B, rem)])

    def zhist(i, c):
        hist_v[pl.ds(i * 16, 16)] = zero16
        return c
    lax.fori_loop(0, SROWS // 16, zhist, 0)
    plsc.subcore_barrier()

    chc = jnp.where(cid == 0, CH0, CH1)
    ebase = jnp.where(cid == 0, sid * (CH0 * B),
                      NS * (CH0 * B) + sid * (CH1 * B))

    def a_descs(c, p):
        base = ebase + c * B
        return (
            pltpu.make_async_copy(row_hbm.at[pl.ds(base, B)], row_v.at[p],
                                  semA[p]),
            pltpu.make_async_copy(col_hbm.at[pl.ds(base, B)], col_v.at[p],
                                  semA[p]),
            pltpu.make_async_copy(ea_hbm.at[pl.ds(base, B)], work_v.at[p],
                                  semA[p]),
        )

    def issue_a(c, p):
        base = ebase + c * B
        pltpu.async_copy(row_hbm.at[pl.ds(base, B)], row_v.at[p], semA[p])
        pltpu.async_copy(col_hbm.at[pl.ds(base, B)], col_v.at[p], semA[p])
        pltpu.async_copy(ea_hbm.at[pl.ds(base, B)], work_v.at[p], semA[p])

    def wait_a(c, p):
        for d in a_descs(c, p):
            d.wait()

    def issue_g(p):
        pltpu.async_copy(xa_hbm.at[row_v.at[p]], work_v.at[p],
                         semG[p], add=True)

    def wait_g(p):
        pltpu.make_async_copy(xa_hbm.at[row_v.at[p]], work_v.at[p],
                              semG[p]).wait()

    def wait_s(p):
        pltpu.make_async_copy(work_v.at[p], acc_sh.at[col_v.at[p]],
                              semS[p]).wait()

    def relu_scatter(p):
        def relu_row(b, cc):
            for j in range(H // 16):
                v = work_v[p, b, pl.ds(j * 16, 16)]
                work_v[p, b, pl.ds(j * 16, 16)] = jnp.maximum(v, 0.0)
            return cc
        lax.fori_loop(0, B, relu_row, 0)
        pltpu.async_copy(work_v.at[p], acc_sh.at[col_v.at[p]],
                         semS[p], add=True)
        ones16 = jnp.full((16,), 1.0, jnp.float32)
        for k in range(B // 16):
            col16 = col_v[p, pl.ds(k * 16, 16)]
            plsc.addupdate_scatter(hist_v, [col16], ones16)

    issue_a(0, 0)
    issue_a(1, 1)
    issue_a(2, 2)
    wait_a(0, 0)
    issue_g(0)
    wait_a(1, 1)
    issue_g(1)

    def step(c, p):
        @pl.when(c + 2 <= chc - 1)
        def _():
            wait_a(c + 2, (p + 2) % NBUF)
            issue_g((p + 2) % NBUF)

        wait_g(p)
        relu_scatter(p)

        @pl.when(c >= 1)
        def _():
            wait_s((p - 1) % NBUF)

        @pl.when(c + 3 <= chc - 1)
        def _():
            issue_a(c + 3, (p + 3) % NBUF)

    def quad(g, carry):
        for k in range(NBUF):
            step(NBUF * g + k, k)
        return carry
    lax.fori_loop(0, chc // NBUF, quad, 0)
    wait_s((CH0 - 1) % NBUF)

    plsc.subcore_barrier()
    r0 = sid * RPW
    pltpu.sync_copy(acc_sh.at[pl.ds(r0, RPW)], out_hbm.at[cid, pl.ds(r0, RPW)])
    pltpu.sync_copy(hist_v, cnt_hbm.at[wid])


def _sc_scatter(xa, ea_pad, row_pad, col_pad):
    mesh = plsc.VectorSubcoreMesh(core_axis_name="c", subcore_axis_name="s")
    kern = pl.kernel(
        _sc_body,
        out_type=(
            jax.ShapeDtypeStruct((NC, SROWS, H), jnp.float32),
            jax.ShapeDtypeStruct((NW, SROWS), jnp.float32),
        ),
        mesh=mesh,
        scratch_types=[
            pltpu.VMEM((NBUF, B), jnp.int32),
            pltpu.VMEM((NBUF, B), jnp.int32),
            pltpu.VMEM((NBUF, B, H), jnp.float32),
            pltpu.VMEM((SROWS,), jnp.float32),
            pltpu.VMEM_SHARED((SROWS, H), jnp.float32),
        ] + [pltpu.SemaphoreType.DMA] * (3 * NBUF),
        compiler_params=pltpu.CompilerParams(needs_layout_passes=False),
    )
    return kern(xa, ea_pad, row_pad, col_pad)


def _post_body(s_ref, c_ref, x_ref, b_ref, u_ref, w2_ref, b2_ref,
               w3a_ref, w3b_ref, w3c_ref, b3_ref, w4_ref, b4_ref, o_ref):
    ssum = s_ref[0] + s_ref[1]
    cnt = jnp.sum(c_ref[:], axis=0)[:, None]
    mean_in = ssum / jnp.maximum(cnt, 1.0)
    mask = (cnt > 0.0).astype(jnp.float32)
    mh = (jnp.dot(mean_in, w2_ref[:], preferred_element_type=jnp.float32)
          + b2_ref[:] * mask)
    uw = jnp.dot(u_ref[:], w3c_ref[:], preferred_element_type=jnp.float32)
    bidx = b_ref[0, 0, :]
    onehot = (bidx[:, None] == lax.iota(jnp.int32, G)[None, :])
    onehot = onehot.astype(jnp.float32)
    pre = (jnp.dot(x_ref[:], w3a_ref[:], preferred_element_type=jnp.float32)
           + jnp.dot(mh, w3b_ref[:], preferred_element_type=jnp.float32)
           + jnp.dot(onehot, uw, preferred_element_type=jnp.float32)
           + b3_ref[:])
    o_ref[:] = (jnp.dot(jnp.maximum(pre, 0.0), w4_ref[:],
                        preferred_element_type=jnp.float32) + b4_ref[:])


def _tc_post(s_acc, cnt_acc, x_pad, batch_pad, u,
             w2, b2, w3a, w3b, w3c, b3, w4, b4):
    grid = SROWS // NB
    return pl.pallas_call(
        _post_body,
        grid=(grid,),
        in_specs=[
            pl.BlockSpec((NC, NB, H), lambda i: (0, i, 0)),
            pl.BlockSpec((NW, NB), lambda i: (0, i)),
            pl.BlockSpec((NB, D), lambda i: (i, 0)),
            pl.BlockSpec((1, 1, NB), lambda i: (i, 0, 0)),
            pl.BlockSpec((G, DG), lambda i: (0, 0)),
            pl.BlockSpec((H, H), lambda i: (0, 0)),
            pl.BlockSpec((1, H), lambda i: (0, 0)),
            pl.BlockSpec((D, H), lambda i: (0, 0)),
            pl.BlockSpec((H, H), lambda i: (0, 0)),
            pl.BlockSpec((DG, H), lambda i: (0, 0)),
            pl.BlockSpec((1, H), lambda i: (0, 0)),
            pl.BlockSpec((H, D), lambda i: (0, 0)),
            pl.BlockSpec((1, D), lambda i: (0, 0)),
        ],
        out_specs=pl.BlockSpec((NB, D), lambda i: (i, 0)),
        out_shape=jax.ShapeDtypeStruct((SROWS, D), jnp.float32),
    )(s_acc, cnt_acc, x_pad, batch_pad, u, w2, b2.reshape(1, H),
      w3a, w3b, w3c, b3.reshape(1, H), w4, b4.reshape(1, D))


@jax.jit
def kernel(x, edge_index, edge_attr, u, batch, W1, b1, W2, b2, W3, b3, W4, b4):
    row = edge_index[0].astype(jnp.int32)
    col = edge_index[1].astype(jnp.int32)
    w1a, w1b = W1[:D], W1[D:]
    w3a, w3b, w3c = W3[:D], W3[D:D + H], W3[D + H:]

    xa = _tc_xa(x, w1a)

    pad = E_PAD - E
    ea = _tc_ea(edge_attr, w1b, b1)
    row_pad = jnp.pad(row, (0, pad))
    col_pad = jnp.pad(col, (0, pad), constant_values=N)

    s_acc, cnt_acc = _sc_scatter(xa, ea, row_pad, col_pad)

    x_pad = jnp.pad(x, ((0, SROWS - N), (0, 0)))
    batch_pad = jnp.pad(batch.astype(jnp.int32), (0, SROWS - N))
    batch_pad = batch_pad.reshape(SROWS // NB, 1, NB)

    out = _tc_post(s_acc, cnt_acc, x_pad, batch_pad, u,
                   W2, b2, w3a, w3b, w3c, b3, W4, b4)
    return out[:N]


# speedup vs baseline: 1.0515x; 1.0011x over previous
import jax
import jax.numpy as jnp
from jax import lax
from jax.experimental import pallas as pl
from jax.experimental.pallas import tpu as pltpu
from jax.experimental.pallas import tpu_sc as plsc

N = 10000
E = 320000
D = 128
DE = 16
H = 128
DG = 128
G = 16

NC = 2
NS = 16
NW = NC * NS
B = 64
CH0 = 224
CH1 = 92
E_PAD = NS * (CH0 + CH1) * B
SROWS = 10240
RPW = SROWS // NS
EB = 6400
NBUF = 4
NB = 1024


def _xa_body(x_ref, w_ref, o_ref):
    o_ref[:] = jnp.dot(x_ref[:], w_ref[:], preferred_element_type=jnp.float32)


def _tc_xa(x, w1a):
    return pl.pallas_call(
        _xa_body,
        out_shape=jax.ShapeDtypeStruct((N, H), jnp.float32),
    )(x, w1a)


def _ea_body(e_ref, w_ref, b_ref, o_ref):
    o_ref[:] = (lax.dot_general(e_ref[:], w_ref[:], (((0,), (0,)), ((), ())),
                                preferred_element_type=jnp.float32)
                + b_ref[:])


def _tc_ea(edge_attr, w1b, b1):
    nreal = E // EB
    nblk = -(-E_PAD // EB)
    return pl.pallas_call(
        _ea_body,
        grid=(nblk,),
        in_specs=[
            pl.BlockSpec((DE, EB), lambda i: (0, jnp.minimum(i, nreal - 1))),
            pl.BlockSpec((DE, H), lambda i: (0, 0)),
            pl.BlockSpec((1, H), lambda i: (0, 0)),
        ],
        out_specs=pl.BlockSpec((EB, H), lambda i: (i, 0)),
        out_shape=jax.ShapeDtypeStruct((nblk * EB, H), jnp.float32),
    )(edge_attr.T, w1b, b1.reshape(1, H))


def _sc_body(xa_hbm, ea_hbm, row_hbm, col_hbm, out_hbm, cnt_hbm,
             row_v, col_v, work_v, hist_v, acc_sh, *sems):
    cid = lax.axis_index("c")
    sid = lax.axis_index("s")
    wid = sid * NC + cid
    semA = sems[0:NBUF]
    semG = sems[NBUF:2 * NBUF]
    semS = sems[2 * NBUF:3 * NBUF]

    zero16 = jnp.zeros((16,), jnp.float32)

    def zrow(b, c):
        for j in range(H // 16):
            work_v[0, b, pl.ds(j * 16, 16)] = zero16
        return c
    lax.fori_loop(0, B, zrow, 0)
    for t in range(RPW // B):
        pltpu.sync_copy(work_v.at[0], acc_sh.at[pl.ds(sid * RPW + t * B, B)])
    rem = RPW % B
    if rem:
        pltpu.sync_copy(
            work_v.at[0, pl.ds(0, rem)],
            acc_sh.at[pl.ds(sid * RPW + (RPW // B) * B, rem)])

    def zhist(i, c):
        hist_v[pl.ds(i * 16, 16)] = zero16
        return c
    lax.fori_loop(0, SROWS // 16, zhist, 0)
    plsc.subcore_barrier()

    chc = jnp.where(cid == 0, CH0, CH1)
    ebase = jnp.where(cid == 0, sid * (CH0 * B),
                      NS * (CH0 * B) + sid * (CH1 * B))

    def a_descs(c, p):
        base = ebase + c * B
        return (
            pltpu.make_async_copy(row_hbm.at[pl.ds(base, B)], row_v.at[p],
                                  semA[p]),
            pltpu.make_async_copy(col_hbm.at[pl.ds(base, B)], col_v.at[p],
                                  semA[p]),
            pltpu.make_async_copy(ea_hbm.at[pl.ds(base, B)], work_v.at[p],
                                  semA[p]),
        )

    def issue_a(c, p):
        base = ebase + c * B
        pltpu.async_copy(row_hbm.at[pl.ds(base, B)], row_v.at[p], semA[p])
        pltpu.async_copy(col_hbm.at[pl.ds(base, B)], col_v.at[p], semA[p])
        pltpu.async_copy(ea_hbm.at[pl.ds(base, B)], work_v.at[p], semA[p])

    def wait_a(c, p):
        for d in a_descs(c, p):
            d.wait()

    def issue_g(p):
        pltpu.async_copy(xa_hbm.at[row_v.at[p]], work_v.at[p],
                         semG[p], add=True)

    def wait_g(p):
        pltpu.make_async_copy(xa_hbm.at[row_v.at[p]], work_v.at[p],
                              semG[p]).wait()

    def wait_s(p):
        pltpu.make_async_copy(work_v.at[p], acc_sh.at[col_v.at[p]],
                              semS[p]).wait()

    def relu_scatter(p):
        def relu_row(b, cc):
            for j in range(H // 16):
                v = work_v[p, b, pl.ds(j * 16, 16)]
                work_v[p, b, pl.ds(j * 16, 16)] = jnp.maximum(v, 0.0)
            return cc
        lax.fori_loop(0, B, relu_row, 0)
        pltpu.async_copy(work_v.at[p], acc_sh.at[col_v.at[p]],
                         semS[p], add=True)
        ones16 = jnp.full((16,), 1.0, jnp.float32)
        for k in range(B // 16):
            col16 = col_v[p, pl.ds(k * 16, 16)]
            plsc.addupdate_scatter(hist_v, [col16], ones16)

    issue_a(0, 0)
    issue_a(1, 1)
    issue_a(2, 2)
    wait_a(0, 0)
    issue_g(0)
    wait_a(1, 1)
    issue_g(1)

    def step(c, p):
        @pl.when(c + 2 <= chc - 1)
        def _():
            wait_a(c + 2, (p + 2) % NBUF)
            issue_g((p + 2) % NBUF)

        wait_g(p)
        relu_scatter(p)

        @pl.when(c >= 1)
        def _():
            wait_s((p - 1) % NBUF)

        @pl.when(c + 3 <= chc - 1)
        def _():
            issue_a(c + 3, (p + 3) % NBUF)

    def quad(g, carry):
        for k in range(NBUF):
            step(NBUF * g + k, k)
        return carry
    lax.fori_loop(0, chc // NBUF, quad, 0)
    wait_s((CH0 - 1) % NBUF)

    plsc.subcore_barrier()
    r0 = sid * RPW
    pltpu.sync_copy(acc_sh.at[pl.ds(r0, RPW)], out_hbm.at[cid, pl.ds(r0, RPW)])
    pltpu.sync_copy(hist_v, cnt_hbm.at[wid])


def _sc_scatter(xa, ea_pad, row_pad, col_pad):
    mesh = plsc.VectorSubcoreMesh(core_axis_name="c", subcore_axis_name="s")
    kern = pl.kernel(
        _sc_body,
        out_type=(
            jax.ShapeDtypeStruct((NC, SROWS, H), jnp.float32),
            jax.ShapeDtypeStruct((NW, SROWS), jnp.float32),
        ),
        mesh=mesh,
        scratch_types=[
            pltpu.VMEM((NBUF, B), jnp.int32),
            pltpu.VMEM((NBUF, B), jnp.int32),
            pltpu.VMEM((NBUF, B, H), jnp.float32),
            pltpu.VMEM((SROWS,), jnp.float32),
            pltpu.VMEM_SHARED((SROWS, H), jnp.float32),
        ] + [pltpu.SemaphoreType.DMA] * (3 * NBUF),
        compiler_params=pltpu.CompilerParams(needs_layout_passes=False),
    )
    return kern(xa, ea_pad, row_pad, col_pad)


def _post_body(s_ref, c_ref, x_ref, b_ref, u_ref, w2_ref, b2_ref,
               w3a_ref, w3b_ref, w3c_ref, b3_ref, w4_ref, b4_ref, o_ref):
    ssum = s_ref[0] + s_ref[1]
    cnt = jnp.sum(c_ref[:], axis=0)[:, None]
    mean_in = ssum / jnp.maximum(cnt, 1.0)
    mask = (cnt > 0.0).astype(jnp.float32)
    mh = (jnp.dot(mean_in, w2_ref[:], preferred_element_type=jnp.float32)
          + b2_ref[:] * mask)
    uw = jnp.dot(u_ref[:], w3c_ref[:], preferred_element_type=jnp.float32)
    bidx = b_ref[0, 0, :]
    onehot = (bidx[:, None] == lax.iota(jnp.int32, G)[None, :])
    onehot = onehot.astype(jnp.float32)
    pre = (jnp.dot(x_ref[:], w3a_ref[:], preferred_element_type=jnp.float32)
           + jnp.dot(mh, w3b_ref[:], preferred_element_type=jnp.float32)
           + jnp.dot(onehot, uw, preferred_element_type=jnp.float32)
           + b3_ref[:])
    o_ref[:] = (jnp.dot(jnp.maximum(pre, 0.0), w4_ref[:],
                        preferred_element_type=jnp.float32) + b4_ref[:])


def _tc_post(s_acc, cnt_acc, x_pad, batch_pad, u,
             w2, b2, w3a, w3b, w3c, b3, w4, b4):
    grid = SROWS // NB
    return pl.pallas_call(
        _post_body,
        grid=(grid,),
        in_specs=[
            pl.BlockSpec((NC, NB, H), lambda i: (0, i, 0)),
            pl.BlockSpec((NW, NB), lambda i: (0, i)),
            pl.BlockSpec((NB, D), lambda i: (i, 0)),
            pl.BlockSpec((1, 1, NB), lambda i: (i, 0, 0)),
            pl.BlockSpec((G, DG), lambda i: (0, 0)),
            pl.BlockSpec((H, H), lambda i: (0, 0)),
            pl.BlockSpec((1, H), lambda i: (0, 0)),
            pl.BlockSpec((D, H), lambda i: (0, 0)),
            pl.BlockSpec((H, H), lambda i: (0, 0)),
            pl.BlockSpec((DG, H), lambda i: (0, 0)),
            pl.BlockSpec((1, H), lambda i: (0, 0)),
            pl.BlockSpec((H, D), lambda i: (0, 0)),
            pl.BlockSpec((1, D), lambda i: (0, 0)),
        ],
        out_specs=pl.BlockSpec((NB, D), lambda i: (i, 0)),
        out_shape=jax.ShapeDtypeStruct((SROWS, D), jnp.float32),
    )(s_acc, cnt_acc, x_pad, batch_pad, u, w2, b2.reshape(1, H),
      w3a, w3b, w3c, b3.reshape(1, H), w4, b4.reshape(1, D))


@jax.jit
def kernel(x, edge_index, edge_attr, u, batch, W1, b1, W2, b2, W3, b3, W4, b4):
    row = edge_index[0].astype(jnp.int32)
    col = edge_index[1].astype(jnp.int32)
    w1a, w1b = W1[:D], W1[D:]
    w3a, w3b, w3c = W3[:D], W3[D:D + H], W3[D + H:]

    xa = _tc_xa(x, w1a)

    pad = E_PAD - E
    ea = _tc_ea(edge_attr, w1b, b1)
    row_pad = jnp.pad(row, (0, pad))
    col_pad = jnp.pad(col, (0, pad), constant_values=N)

    s_acc, cnt_acc = _sc_scatter(xa, ea, row_pad, col_pad)

    x_pad = jnp.pad(x, ((0, SROWS - N), (0, 0)))
    batch_pad = jnp.pad(batch.astype(jnp.int32), (0, SROWS - N))
    batch_pad = batch_pad.reshape(SROWS // NB, 1, NB)

    out = _tc_post(s_acc, cnt_acc, x_pad, batch_pad, u,
                   W2, b2, w3a, w3b, w3c, b3, W4, b4)
    return out[:N]
